# baseline (speedup 1.0000x reference)
import functools

import numpy as np
import jax
import jax.numpy as jnp
from jax import lax
from jax.experimental import pallas as pl
from jax.experimental.pallas import tpu as pltpu
from jax.experimental.pallas import tpu_sc as plsc

_VOCAB = 1000000
_EMB = 64
_SEQ = 200
_BATCH = 1024

_NC = 2
_NS = 16
_L = 16
_NW = _NC * _NS

_SEQ_PER_CHUNK = 2
_R = _SEQ_PER_CHUNK * _SEQ
_TOT = _BATCH * _SEQ
_BPW = _TOT // _NW
_NCHUNK = _BPW // _R
_GSPLIT = 4
_GROWS = _R // _GSPLIT


def _pos_encoding():
    pos = np.arange(_SEQ, dtype=np.float32)[:, None]
    i = np.arange(0, _EMB, 2, dtype=np.float32)
    div = np.power(10000.0, i / _EMB)
    pe = np.zeros((_SEQ, _EMB), np.float32)
    pe[:, 0::2] = np.sin(pos / div)
    pe[:, 1::2] = np.cos(pos / div)
    return pe


_PE = _pos_encoding()

_mesh = plsc.VectorSubcoreMesh(core_axis_name="c", subcore_axis_name="s")


@functools.partial(
    pl.kernel,
    mesh=_mesh,
    out_type=jax.ShapeDtypeStruct((_TOT, _EMB), jnp.float32),
    compiler_params=pltpu.CompilerParams(use_tc_tiling_on_sc=False),
    scratch_types=[
        pltpu.VMEM((_NCHUNK * _GSPLIT, _GROWS), jnp.int32),
        pltpu.VMEM((_SEQ, _EMB), jnp.float32),
        pltpu.VMEM((_R, _EMB), jnp.float32),
        pltpu.VMEM((_R, _EMB), jnp.float32),
        pltpu.SemaphoreType.DMA,
        pltpu.SemaphoreType.DMA,
        pltpu.SemaphoreType.DMA,
        pltpu.SemaphoreType.DMA,
    ],
)
def _emb_kernel(x_hbm, table_hbm, pe_hbm, out_hbm,
                idx_v, pe_v, buf0, buf1, gsem0, gsem1, osem0, osem1):
    wid = lax.axis_index("s") * _NC + lax.axis_index("c")
    base = wid * _BPW

    pltpu.sync_copy(x_hbm.at[pl.ds(wid * _NCHUNK * _GSPLIT, _NCHUNK * _GSPLIT)],
                    idx_v)
    pltpu.sync_copy(pe_hbm, pe_v)

    bufs = (buf0, buf1)
    gsems = (gsem0, gsem1)
    osems = (osem0, osem1)

    def fire_gather(c, p):
        handles = []
        for j in range(_GSPLIT):
            handles.append(pltpu.async_copy(
                table_hbm.at[idx_v.at[c * _GSPLIT + j]],
                bufs[p].at[pl.ds(j * _GROWS, _GROWS)],
                gsems[p]))
        return handles

    gh = [None, None]
    oh = [None, None]
    gh[0] = fire_gather(0, 0)

    for c in range(_NCHUNK):
        p = c & 1
        if c + 1 < _NCHUNK:
            pn = 1 - p
            if oh[pn] is not None:
                oh[pn].wait()
                oh[pn] = None
            gh[pn] = fire_gather(c + 1, pn)
        for h in gh[p]:
            h.wait()
        buf = bufs[p]

        def add_pe(q, carry):
            for k in range(_EMB // _L):
                col = k * _L
                pv = pe_v[q, pl.ds(col, _L)]
                for s in range(_SEQ_PER_CHUNK):
                    r = s * _SEQ + q
                    buf[r, pl.ds(col, _L)] += pv
            return carry

        lax.fori_loop(0, _SEQ, add_pe, 0)

        oh[p] = pltpu.async_copy(
            buf, out_hbm.at[pl.ds(base + c * _R, _R)], osems[p])

    for p in range(2):
        if oh[p] is not None:
            oh[p].wait()


def kernel(x, table):
    xf = x.reshape(_NW * _NCHUNK * _GSPLIT, _GROWS).astype(jnp.int32)
    pe = jnp.asarray(_PE)
    out = _emb_kernel(xf, table, pe)
    return out.reshape(_BATCH, _SEQ, _EMB)


# speedup vs baseline: 1.7264x; 1.7264x over previous
import functools

import numpy as np
import jax
import jax.numpy as jnp
from jax import lax
from jax.experimental import pallas as pl
from jax.experimental.pallas import tpu as pltpu
from jax.experimental.pallas import tpu_sc as plsc

_VOCAB = 1000000
_EMB = 64
_SEQ = 200
_BATCH = 1024

_NC = 2
_NS = 16
_L = 16
_NW = _NC * _NS

_SEQ_PER_CHUNK = 2
_R = _SEQ_PER_CHUNK * _SEQ
_TOT = _BATCH * _SEQ
_BPW = _TOT // _NW
_NCHUNK = _BPW // _R
_GSPLIT = 5
_GROWS = _R // _GSPLIT

_P = 512000
_MB = 4096


def _pos_encoding():
    pos = np.arange(_SEQ, dtype=np.float32)[:, None]
    i = np.arange(0, _EMB, 2, dtype=np.float32)
    div = np.power(10000.0, i / _EMB)
    pe = np.zeros((_SEQ, _EMB), np.float32)
    pe[:, 0::2] = np.sin(pos / div)
    pe[:, 1::2] = np.cos(pos / div)
    return pe


_PE = _pos_encoding()


def _pack_body(in0_ref, in1_ref, o_ref):
    i0 = lax.broadcasted_iota(jnp.int32, (_EMB, _EMB), 0)
    i1 = lax.broadcasted_iota(jnp.int32, (_EMB, _EMB), 1)
    ident = (i0 == i1).astype(jnp.float32)
    dn = (((0,), (0,)), ((), ()))
    o_ref[:, 0:_EMB] = lax.dot_general(
        in0_ref[...], ident, dn, preferred_element_type=jnp.float32)
    o_ref[:, _EMB:2 * _EMB] = lax.dot_general(
        in1_ref[...], ident, dn, preferred_element_type=jnp.float32)


_N_IN_BLOCKS = _VOCAB // _MB

_pack = pl.pallas_call(
    _pack_body,
    grid=(_P // _MB,),
    in_specs=[
        pl.BlockSpec((_EMB, _MB), lambda q: (0, q)),
        pl.BlockSpec((_EMB, _MB),
                     lambda q: (0, jnp.minimum(q + _P // _MB, _N_IN_BLOCKS))),
    ],
    out_specs=pl.BlockSpec((_MB, 2 * _EMB), lambda q: (q, 0)),
    out_shape=jax.ShapeDtypeStruct((_P, 2 * _EMB), jnp.float32),
)

_mesh = plsc.VectorSubcoreMesh(core_axis_name="c", subcore_axis_name="s")


@functools.partial(
    pl.kernel,
    mesh=_mesh,
    out_type=jax.ShapeDtypeStruct((_TOT, _EMB), jnp.float32),
    compiler_params=pltpu.CompilerParams(use_tc_tiling_on_sc=False),
    scratch_types=[
        pltpu.VMEM((_BPW,), jnp.int32),
        pltpu.VMEM((_SEQ, _EMB), jnp.float32),
        pltpu.VMEM((_R, _EMB), jnp.float32),
        pltpu.VMEM((_R, _EMB), jnp.float32),
        pltpu.SemaphoreType.DMA,
        pltpu.SemaphoreType.DMA,
        pltpu.SemaphoreType.DMA,
        pltpu.SemaphoreType.DMA,
    ],
)
def _emb_kernel(x_hbm, table_hbm, pe_hbm, out_hbm,
                idx_v, pe_v, buf0, buf1, gsem0, gsem1, osem0, osem1):
    wid = lax.axis_index("s") * _NC + lax.axis_index("c")
    base = wid * _BPW

    pltpu.sync_copy(x_hbm.at[pl.ds(base, _BPW)], idx_v)
    pltpu.sync_copy(pe_hbm, pe_v)

    def remap(k, carry):
        v = idx_v[pl.ds(k * _L, _L)]
        idx_v[pl.ds(k * _L, _L)] = jnp.where(
            v >= _P, 2 * (v - _P) + 1, 2 * v)
        return carry

    lax.fori_loop(0, _BPW // _L, remap, 0)

    bufs = (buf0, buf1)
    gsems = (gsem0, gsem1)
    osems = (osem0, osem1)

    def fire_gather(c, p):
        handles = []
        for j in range(_GSPLIT):
            handles.append(pltpu.async_copy(
                table_hbm.at[idx_v.at[pl.ds(c * _R + j * _GROWS, _GROWS)]],
                bufs[p].at[pl.ds(j * _GROWS, _GROWS)],
                gsems[p]))
        return handles

    gh = [None, None]
    oh = [None, None]
    gh[0] = fire_gather(0, 0)

    for c in range(_NCHUNK):
        p = c & 1
        if c + 1 < _NCHUNK:
            pn = 1 - p
            if oh[pn] is not None:
                oh[pn].wait()
                oh[pn] = None
            gh[pn] = fire_gather(c + 1, pn)
        for h in gh[p]:
            h.wait()
        buf = bufs[p]

        def add_pe(q, carry):
            for k in range(_EMB // _L):
                col = k * _L
                pv = pe_v[q, pl.ds(col, _L)]
                for s in range(_SEQ_PER_CHUNK):
                    r = s * _SEQ + q
                    buf[r, pl.ds(col, _L)] += pv
            return carry

        lax.fori_loop(0, _SEQ, add_pe, 0)

        oh[p] = pltpu.async_copy(
            buf, out_hbm.at[pl.ds(base + c * _R, _R)], osems[p])

    for p in range(2):
        if oh[p] is not None:
            oh[p].wait()


def kernel(x, table):
    xf = x.reshape(-1).astype(jnp.int32)
    pe = jnp.asarray(_PE)
    tt = table.T
    packed = _pack(tt, tt)
    t2v = packed.reshape(2 * _P, _EMB)
    out = _emb_kernel(xf, t2v, pe)
    return out.reshape(_BATCH, _SEQ, _EMB)


# speedup vs baseline: 1.8989x; 1.0999x over previous
import functools

import numpy as np
import jax
import jax.numpy as jnp
from jax import lax
from jax.experimental import pallas as pl
from jax.experimental.pallas import tpu as pltpu
from jax.experimental.pallas import tpu_sc as plsc

_VOCAB = 1000000
_EMB = 64
_SEQ = 200
_BATCH = 1024

_NC = 2
_NS = 16
_L = 16
_NW = _NC * _NS

_P = 512000
_MB = 4096

_NUNITS = (_SEQ // 8) * 8 * 8
_UPW = _NUNITS // _NW


def _pos_encoding():
    pos = np.arange(_SEQ, dtype=np.float32)[:, None]
    i = np.arange(0, _EMB, 2, dtype=np.float32)
    div = np.power(10000.0, i / _EMB)
    pe = np.zeros((_SEQ, _EMB), np.float32)
    pe[:, 0::2] = np.sin(pos / div)
    pe[:, 1::2] = np.cos(pos / div)
    return pe


_PE = _pos_encoding()


def _pack_body(in0_ref, in1_ref, o_ref):
    i0 = lax.broadcasted_iota(jnp.int32, (_EMB, _EMB), 0)
    i1 = lax.broadcasted_iota(jnp.int32, (_EMB, _EMB), 1)
    ident = (i0 == i1).astype(jnp.float32)
    dn = (((0,), (0,)), ((), ()))
    o_ref[:, 0:_EMB] = lax.dot_general(
        in0_ref[...], ident, dn, preferred_element_type=jnp.float32)
    o_ref[:, _EMB:2 * _EMB] = lax.dot_general(
        in1_ref[...], ident, dn, preferred_element_type=jnp.float32)


_N_IN_BLOCKS = _VOCAB // _MB

_pack = pl.pallas_call(
    _pack_body,
    grid=(_P // _MB,),
    in_specs=[
        pl.BlockSpec((_EMB, _MB), lambda q: (0, q)),
        pl.BlockSpec((_EMB, _MB),
                     lambda q: (0, jnp.minimum(q + _P // _MB, _N_IN_BLOCKS))),
    ],
    out_specs=pl.BlockSpec((_MB, 2 * _EMB), lambda q: (q, 0)),
    out_shape=jax.ShapeDtypeStruct((_P, 2 * _EMB), jnp.float32),
)

_mesh = plsc.VectorSubcoreMesh(core_axis_name="c", subcore_axis_name="s")


@functools.partial(
    pl.kernel,
    mesh=_mesh,
    out_type=jax.ShapeDtypeStruct((_SEQ, 8, 8, 8, 128), jnp.float32),
    compiler_params=pltpu.CompilerParams(use_tc_tiling_on_sc=False,
                                         needs_layout_passes=False),
    scratch_types=[
        pltpu.VMEM((_UPW, 128), jnp.int32),
        pltpu.VMEM((_SEQ, _EMB), jnp.float32),
        pltpu.VMEM((128, _EMB), jnp.float32),
        pltpu.VMEM((128, _EMB), jnp.float32),
        pltpu.VMEM((8, 8, 129), jnp.float32),
        pltpu.VMEM((8, 8, 129), jnp.float32),
        pltpu.SemaphoreType.DMA,
        pltpu.SemaphoreType.DMA,
        pltpu.SemaphoreType.DMA,
        pltpu.SemaphoreType.DMA,
    ],
)
def _emb_kernel(x_hbm, table_hbm, pe_hbm, out_hbm,
                idx_v, pe_v, gbuf0, gbuf1, tbuf0, tbuf1,
                gsem0, gsem1, osem0, osem1):
    wid = lax.axis_index("s") * _NC + lax.axis_index("c")
    ubase = wid * _UPW

    pltpu.sync_copy(x_hbm.at[pl.ds(ubase, _UPW)], idx_v)
    pltpu.sync_copy(pe_hbm, pe_v)

    def remap_row(k, carry):
        for c in range(128 // _L):
            v = idx_v[k, pl.ds(c * _L, _L)]
            idx_v[k, pl.ds(c * _L, _L)] = jnp.where(
                v >= _P, 2 * (v - _P) + 1, 2 * v)
        return carry

    lax.fori_loop(0, _UPW, remap_row, 0)

    gbufs = (gbuf0, gbuf1)
    tbufs = (tbuf0, tbuf1)
    gsems = (gsem0, gsem1)
    osems = (osem0, osem1)

    lane = lax.broadcasted_iota(jnp.int32, (_L,), 0)
    e8_l = lane & 7
    je_half = lane >> 3

    def gather_cp(i, p):
        return pltpu.make_async_copy(
            table_hbm.at[idx_v.at[i]], gbufs[p], gsems[p])

    def unit_sb(i):
        v = ubase + i
        s = (v >> 6) * 8 + (v & 7)
        bb = (v >> 3) & 7
        return s, bb

    def out_cps(i, p):
        s, bb = unit_sb(i)
        return [pltpu.make_async_copy(tbufs[p].at[je, :, pl.ds(0, 128)],
                                      out_hbm.at[s, je, bb], osems[p])
                for je in range(8)]

    gather_cp(0, 0).start()
    gather_cp(1, 1).start()

    def pair_body(kk, carry):
        for d in range(2):
            i = 2 * kk + d
            gbuf = gbufs[d]
            tbuf = tbufs[d]
            gather_cp(i, d).wait()

            @pl.when(i >= 2)
            def _():
                for cp in out_cps(i - 2, d):
                    cp.wait()

            s, bb = unit_sb(i)
            pvs = [pe_v[s, pl.ds(jc * _L, _L)] for jc in range(_EMB // _L)]

            def transpose_pe(t, carry2):
                t_v = jnp.zeros((_L,), jnp.int32) + t
                for jc in range(_EMB // _L):
                    val = gbuf[t, pl.ds(jc * _L, _L)] + pvs[jc]
                    plsc.store_scatter(
                        tbuf, [je_half + 2 * jc, e8_l, t_v], val)
                return carry2

            lax.fori_loop(0, 128, transpose_pe, 0)

            for cp in out_cps(i, d):
                cp.start()

            @pl.when(i + 2 < _UPW)
            def _():
                gather_cp(i + 2, d).start()
        return carry

    lax.fori_loop(0, _UPW // 2, pair_body, 0)

    for d in range(2):
        for cp in out_cps(_UPW - 2 + d, d):
            cp.wait()


def kernel(x, table):
    xt4 = x.T.reshape(_SEQ // 8, 8, 8, 128).transpose(0, 2, 1, 3)
    xt4 = xt4.reshape(_NUNITS, 128).astype(jnp.int32)
    pe = jnp.asarray(_PE)
    tt = table.T
    packed = _pack(tt, tt)
    t2v = packed.reshape(2 * _P, _EMB)
    out5 = _emb_kernel(xt4, t2v, pe)
    return out5.transpose(2, 4, 0, 1, 3).reshape(_BATCH, _SEQ, _EMB)


# speedup vs baseline: 1.9219x; 1.0121x over previous
import functools

import numpy as np
import jax
import jax.numpy as jnp
from jax import lax
from jax.experimental import pallas as pl
from jax.experimental.pallas import tpu as pltpu
from jax.experimental.pallas import tpu_sc as plsc

_VOCAB = 1000000
_EMB = 64
_SEQ = 200
_BATCH = 1024

_NC = 2
_NS = 16
_L = 16
_NW = _NC * _NS

_P = 512000
_MB = 4096

_NUNITS = (_SEQ // 8) * 8 * 8
_UPW = _NUNITS // _NW


def _pos_encoding():
    pos = np.arange(_SEQ, dtype=np.float32)[:, None]
    i = np.arange(0, _EMB, 2, dtype=np.float32)
    div = np.power(10000.0, i / _EMB)
    pe = np.zeros((_SEQ, _EMB), np.float32)
    pe[:, 0::2] = np.sin(pos / div)
    pe[:, 1::2] = np.cos(pos / div)
    return pe


_PE = _pos_encoding()


def _pack_body(in0_ref, in1_ref, o_ref):
    i0 = lax.broadcasted_iota(jnp.int32, (_EMB, _EMB), 0)
    i1 = lax.broadcasted_iota(jnp.int32, (_EMB, _EMB), 1)
    ident = (i0 == i1).astype(jnp.float32)
    dn = (((0,), (0,)), ((), ()))
    o_ref[:, 0:_EMB] = in0_ref[...].T
    o_ref[:, _EMB:2 * _EMB] = lax.dot_general(
        in1_ref[...], ident, dn, preferred_element_type=jnp.float32)


_N_IN_BLOCKS = _VOCAB // _MB

_pack = pl.pallas_call(
    _pack_body,
    grid=(_P // _MB,),
    in_specs=[
        pl.BlockSpec((_EMB, _MB), lambda q: (0, q)),
        pl.BlockSpec((_EMB, _MB),
                     lambda q: (0, jnp.minimum(q + _P // _MB, _N_IN_BLOCKS))),
    ],
    out_specs=pl.BlockSpec((_MB, 2 * _EMB), lambda q: (q, 0)),
    out_shape=jax.ShapeDtypeStruct((_P, 2 * _EMB), jnp.float32),
)

_mesh = plsc.VectorSubcoreMesh(core_axis_name="c", subcore_axis_name="s")


@functools.partial(
    pl.kernel,
    mesh=_mesh,
    out_type=jax.ShapeDtypeStruct((_SEQ, 8, 8, 8, 128), jnp.float32),
    compiler_params=pltpu.CompilerParams(use_tc_tiling_on_sc=False,
                                         needs_layout_passes=False),
    scratch_types=[
        pltpu.VMEM((_UPW, 128), jnp.int32),
        pltpu.VMEM((_SEQ, _EMB), jnp.float32),
        pltpu.VMEM((128, _EMB), jnp.float32),
        pltpu.VMEM((128, _EMB), jnp.float32),
        pltpu.VMEM((8, 8, 129), jnp.float32),
        pltpu.VMEM((8, 8, 129), jnp.float32),
        pltpu.SemaphoreType.DMA,
        pltpu.SemaphoreType.DMA,
        pltpu.SemaphoreType.DMA,
        pltpu.SemaphoreType.DMA,
    ],
)
def _emb_kernel(x_hbm, table_hbm, pe_hbm, out_hbm,
                idx_v, pe_v, gbuf0, gbuf1, tbuf0, tbuf1,
                gsem0, gsem1, osem0, osem1):
    wid = lax.axis_index("s") * _NC + lax.axis_index("c")
    ubase = wid * _UPW

    pltpu.sync_copy(x_hbm.at[pl.ds(ubase, _UPW)], idx_v)
    pltpu.sync_copy(pe_hbm, pe_v)

    def remap_row(k, carry):
        for c in range(128 // _L):
            v = idx_v[k, pl.ds(c * _L, _L)]
            idx_v[k, pl.ds(c * _L, _L)] = jnp.where(
                v >= _P, 2 * (v - _P) + 1, 2 * v)
        return carry

    lax.fori_loop(0, _UPW, remap_row, 0)

    gbufs = (gbuf0, gbuf1)
    tbufs = (tbuf0, tbuf1)
    gsems = (gsem0, gsem1)
    osems = (osem0, osem1)

    lane = lax.broadcasted_iota(jnp.int32, (_L,), 0)
    e8_l = lane & 7
    je_half = lane >> 3

    def gather_cp(i, p):
        return pltpu.make_async_copy(
            table_hbm.at[idx_v.at[i]], gbufs[p], gsems[p])

    def unit_sb(i):
        v = ubase + i
        s = (v >> 6) * 8 + (v & 7)
        bb = (v >> 3) & 7
        return s, bb

    def out_cps(i, p):
        s, bb = unit_sb(i)
        return [pltpu.make_async_copy(tbufs[p].at[:, :, pl.ds(0, 128)],
                                      out_hbm.at[s, :, bb], osems[p])]

    gather_cp(0, 0).start()
    gather_cp(1, 1).start()

    def pair_body(kk, carry):
        for d in range(2):
            i = 2 * kk + d
            gbuf = gbufs[d]
            tbuf = tbufs[d]
            gather_cp(i, d).wait()

            @pl.when(i >= 2)
            def _():
                for cp in out_cps(i - 2, d):
                    cp.wait()

            s, bb = unit_sb(i)
            pvs = [pe_v[s, pl.ds(jc * _L, _L)] for jc in range(_EMB // _L)]

            def transpose_pe(t0, carry2):
                t = 4 * t0
                for dt in range(4):
                    t_v = jnp.zeros((_L,), jnp.int32) + (t + dt)
                    for jc in range(_EMB // _L):
                        val = gbuf[t + dt, pl.ds(jc * _L, _L)] + pvs[jc]
                        plsc.store_scatter(
                            tbuf, [je_half + 2 * jc, e8_l, t_v], val)
                return carry2

            lax.fori_loop(0, 32, transpose_pe, 0)

            for cp in out_cps(i, d):
                cp.start()

            @pl.when(i + 2 < _UPW)
            def _():
                gather_cp(i + 2, d).start()
        return carry

    lax.fori_loop(0, _UPW // 2, pair_body, 0)

    for d in range(2):
        for cp in out_cps(_UPW - 2 + d, d):
            cp.wait()


def kernel(x, table):
    xt4 = x.T.reshape(_SEQ // 8, 8, 8, 128).transpose(0, 2, 1, 3)
    xt4 = xt4.reshape(_NUNITS, 128).astype(jnp.int32)
    pe = jnp.asarray(_PE)
    tt = table.T
    packed = _pack(tt, tt)
    t2v = packed.reshape(2 * _P, _EMB)
    out5 = _emb_kernel(xt4, t2v, pe)
    return out5.transpose(2, 4, 0, 1, 3).reshape(_BATCH, _SEQ, _EMB)


# speedup vs baseline: 1.9234x; 1.0008x over previous
import functools

import numpy as np
import jax
import jax.numpy as jnp
from jax import lax
from jax.experimental import pallas as pl
from jax.experimental.pallas import tpu as pltpu
from jax.experimental.pallas import tpu_sc as plsc

_VOCAB = 1000000
_EMB = 64
_SEQ = 200
_BATCH = 1024

_NC = 2
_NS = 16
_L = 16
_NW = _NC * _NS

_P = 512000
_MB = 4096

_NUNITS = (_SEQ // 8) * 8 * 8
_UPW = _NUNITS // _NW


def _pos_encoding():
    pos = np.arange(_SEQ, dtype=np.float32)[:, None]
    i = np.arange(0, _EMB, 2, dtype=np.float32)
    div = np.power(10000.0, i / _EMB)
    pe = np.zeros((_SEQ, _EMB), np.float32)
    pe[:, 0::2] = np.sin(pos / div)
    pe[:, 1::2] = np.cos(pos / div)
    return pe


_PE = _pos_encoding()


def _pack_body(in0_ref, in1_ref, o_ref):
    i0 = lax.broadcasted_iota(jnp.int32, (_EMB, _EMB), 0)
    i1 = lax.broadcasted_iota(jnp.int32, (_EMB, _EMB), 1)
    ident = (i0 == i1).astype(jnp.float32)
    dn = (((0,), (0,)), ((), ()))
    t0 = in0_ref[...].T
    t1 = lax.dot_general(
        in1_ref[...], ident, dn, preferred_element_type=jnp.float32)
    o_ref[...] = jnp.concatenate([t0, t1], axis=1)


_N_IN_BLOCKS = _VOCAB // _MB

_pack = pl.pallas_call(
    _pack_body,
    grid=(_P // _MB,),
    in_specs=[
        pl.BlockSpec((_EMB, _MB), lambda q: (0, q)),
        pl.BlockSpec((_EMB, _MB),
                     lambda q: (0, jnp.minimum(q + _P // _MB, _N_IN_BLOCKS))),
    ],
    out_specs=pl.BlockSpec((_MB, 2 * _EMB), lambda q: (q, 0)),
    out_shape=jax.ShapeDtypeStruct((_P, 2 * _EMB), jnp.float32),
)

_mesh = plsc.VectorSubcoreMesh(core_axis_name="c", subcore_axis_name="s")


@functools.partial(
    pl.kernel,
    mesh=_mesh,
    out_type=jax.ShapeDtypeStruct((_SEQ, 8, 8, 8, 128), jnp.float32),
    compiler_params=pltpu.CompilerParams(use_tc_tiling_on_sc=False,
                                         needs_layout_passes=False),
    scratch_types=[
        pltpu.VMEM((_UPW, 128), jnp.int32),
        pltpu.VMEM((_SEQ, _EMB), jnp.float32),
        pltpu.VMEM((128, _EMB), jnp.float32),
        pltpu.VMEM((128, _EMB), jnp.float32),
        pltpu.VMEM((8, 8, 129), jnp.float32),
        pltpu.VMEM((8, 8, 129), jnp.float32),
        pltpu.SemaphoreType.DMA,
        pltpu.SemaphoreType.DMA,
        pltpu.SemaphoreType.DMA,
        pltpu.SemaphoreType.DMA,
    ],
)
def _emb_kernel(x_hbm, table_hbm, pe_hbm, out_hbm,
                idx_v, pe_v, gbuf0, gbuf1, tbuf0, tbuf1,
                gsem0, gsem1, osem0, osem1):
    wid = lax.axis_index("s") * _NC + lax.axis_index("c")
    ubase = wid * _UPW

    pltpu.sync_copy(x_hbm.at[pl.ds(ubase, _UPW)], idx_v)
    pltpu.sync_copy(pe_hbm, pe_v)

    def remap_row(k, carry):
        for c in range(128 // _L):
            v = idx_v[k, pl.ds(c * _L, _L)]
            idx_v[k, pl.ds(c * _L, _L)] = jnp.where(
                v >= _P, 2 * (v - _P) + 1, 2 * v)
        return carry

    lax.fori_loop(0, _UPW, remap_row, 0)

    gbufs = (gbuf0, gbuf1)
    tbufs = (tbuf0, tbuf1)
    gsems = (gsem0, gsem1)
    osems = (osem0, osem1)

    lane = lax.broadcasted_iota(jnp.int32, (_L,), 0)
    e8_l = lane & 7
    je_half = lane >> 3
    jidx = [je_half + 2 * jc for jc in range(_EMB // _L)]
    ones_v = jnp.full((_L,), 1, jnp.int32)
    zeros_v = jnp.zeros((_L,), jnp.int32)

    def gather_cp(i, p):
        return pltpu.make_async_copy(
            table_hbm.at[idx_v.at[i]], gbufs[p], gsems[p])

    def unit_sb(i):
        v = ubase + i
        s = (v >> 6) * 8 + (v & 7)
        bb = (v >> 3) & 7
        return s, bb

    def out_cps(i, p):
        s, bb = unit_sb(i)
        return [pltpu.make_async_copy(tbufs[p].at[:, :, pl.ds(0, 128)],
                                      out_hbm.at[s, :, bb], osems[p])]

    gather_cp(0, 0).start()
    gather_cp(1, 1).start()

    def pair_body(kk, carry):
        for d in range(2):
            i = 2 * kk + d
            gbuf = gbufs[d]
            tbuf = tbufs[d]
            gather_cp(i, d).wait()

            @pl.when(i >= 2)
            def _():
                for cp in out_cps(i - 2, d):
                    cp.wait()

            s, bb = unit_sb(i)
            pvs = [pe_v[s, pl.ds(jc * _L, _L)] for jc in range(_EMB // _L)]

            def transpose_pe(t0, t_v):
                t = 4 * t0
                for dt in range(4):
                    for jc in range(_EMB // _L):
                        val = gbuf[t + dt, pl.ds(jc * _L, _L)] + pvs[jc]
                        plsc.store_scatter(
                            tbuf, [jidx[jc], e8_l, t_v], val)
                    t_v = t_v + ones_v
                return t_v

            lax.fori_loop(0, 32, transpose_pe, zeros_v)

            for cp in out_cps(i, d):
                cp.start()

            @pl.when(i + 2 < _UPW)
            def _():
                gather_cp(i + 2, d).start()
        return carry

    lax.fori_loop(0, _UPW // 2, pair_body, 0)

    for d in range(2):
        for cp in out_cps(_UPW - 2 + d, d):
            cp.wait()


def kernel(x, table):
    xt4 = x.T.reshape(_SEQ // 8, 8, 8, 128).transpose(0, 2, 1, 3)
    xt4 = xt4.reshape(_NUNITS, 128).astype(jnp.int32)
    pe = jnp.asarray(_PE)
    tt = table.T
    packed = _pack(tt, tt)
    t2v = packed.reshape(2 * _P, _EMB)
    out5 = _emb_kernel(xt4, t2v, pe)
    return out5.transpose(2, 4, 0, 1, 3).reshape(_BATCH, _SEQ, _EMB)


# speedup vs baseline: 2.0402x; 1.0607x over previous
import functools

import numpy as np
import jax
import jax.numpy as jnp
from jax import lax
from jax.experimental import pallas as pl
from jax.experimental.pallas import tpu as pltpu
from jax.experimental.pallas import tpu_sc as plsc

_VOCAB = 1000000
_EMB = 64
_SEQ = 200
_BATCH = 1024

_NC = 2
_NS = 16
_L = 16
_NW = _NC * _NS

_P = 512000
_MB = 6400

_NUNITS = (_SEQ // 8) * 8 * 8
_UPW = _NUNITS // _NW


def _pos_encoding():
    pos = np.arange(_SEQ, dtype=np.float32)[:, None]
    i = np.arange(0, _EMB, 2, dtype=np.float32)
    div = np.power(10000.0, i / _EMB)
    pe = np.zeros((_SEQ, _EMB), np.float32)
    pe[:, 0::2] = np.sin(pos / div)
    pe[:, 1::2] = np.cos(pos / div)
    return pe


_PE = _pos_encoding()


def _pack_body(in0_ref, in1_ref, o_ref):
    i0 = lax.broadcasted_iota(jnp.int32, (_EMB, _EMB), 0)
    i1 = lax.broadcasted_iota(jnp.int32, (_EMB, _EMB), 1)
    ident = (i0 == i1).astype(jnp.float32)
    dn = (((0,), (0,)), ((), ()))
    t0 = in0_ref[...].T
    t1 = lax.dot_general(
        in1_ref[...], ident, dn, preferred_element_type=jnp.float32)
    o_ref[...] = jnp.concatenate([t0, t1], axis=1)


_N_IN_BLOCKS = _VOCAB // _MB

_pack = pl.pallas_call(
    _pack_body,
    grid=(_P // _MB,),
    in_specs=[
        pl.BlockSpec((_EMB, _MB), lambda q: (0, q)),
        pl.BlockSpec((_EMB, _MB),
                     lambda q: (0, jnp.minimum(q + _P // _MB, _N_IN_BLOCKS))),
    ],
    out_specs=pl.BlockSpec((_MB, 2 * _EMB), lambda q: (q, 0)),
    out_shape=jax.ShapeDtypeStruct((_P, 2 * _EMB), jnp.float32),
)

_mesh = plsc.VectorSubcoreMesh(core_axis_name="c", subcore_axis_name="s")


@functools.partial(
    pl.kernel,
    mesh=_mesh,
    out_type=jax.ShapeDtypeStruct((_SEQ, 8, 8, 8, 128), jnp.float32),
    compiler_params=pltpu.CompilerParams(use_tc_tiling_on_sc=False,
                                         needs_layout_passes=False),
    scratch_types=[
        pltpu.VMEM((_UPW, 128), jnp.int32),
        pltpu.VMEM((_SEQ, _EMB), jnp.float32),
        pltpu.VMEM((128, _EMB), jnp.float32),
        pltpu.VMEM((128, _EMB), jnp.float32),
        pltpu.VMEM((8, 8, 129), jnp.float32),
        pltpu.VMEM((8, 8, 129), jnp.float32),
        pltpu.SemaphoreType.DMA,
        pltpu.SemaphoreType.DMA,
        pltpu.SemaphoreType.DMA,
        pltpu.SemaphoreType.DMA,
    ],
)
def _emb_kernel(x_hbm, table_hbm, pe_hbm, out_hbm,
                idx_v, pe_v, gbuf0, gbuf1, tbuf0, tbuf1,
                gsem0, gsem1, osem0, osem1):
    wid = lax.axis_index("s") * _NC + lax.axis_index("c")
    ubase = wid * _UPW

    pltpu.sync_copy(x_hbm.at[pl.ds(ubase, _UPW)], idx_v)
    pltpu.sync_copy(pe_hbm, pe_v)

    def remap_row(k, carry):
        for c in range(128 // _L):
            v = idx_v[k, pl.ds(c * _L, _L)]
            idx_v[k, pl.ds(c * _L, _L)] = jnp.where(
                v >= _P, 2 * (v - _P) + 1, 2 * v)
        return carry

    lax.fori_loop(0, _UPW, remap_row, 0)

    gbufs = (gbuf0, gbuf1)
    tbufs = (tbuf0, tbuf1)
    gsems = (gsem0, gsem1)
    osems = (osem0, osem1)

    lane = lax.broadcasted_iota(jnp.int32, (_L,), 0)
    e8_l = lane & 7
    je_half = lane >> 3
    jidx = [je_half + 2 * jc for jc in range(_EMB // _L)]
    ones_v = jnp.full((_L,), 1, jnp.int32)
    zeros_v = jnp.zeros((_L,), jnp.int32)

    def gather_cp(i, p):
        return pltpu.make_async_copy(
            table_hbm.at[idx_v.at[i]], gbufs[p], gsems[p])

    def unit_sb(i):
        v = ubase + i
        s = (v >> 6) * 8 + (v & 7)
        bb = (v >> 3) & 7
        return s, bb

    def out_cps(i, p):
        s, bb = unit_sb(i)
        return [pltpu.make_async_copy(tbufs[p].at[:, :, pl.ds(0, 128)],
                                      out_hbm.at[s, :, bb], osems[p])]

    gather_cp(0, 0).start()
    gather_cp(1, 1).start()

    def pair_body(kk, carry):
        for d in range(2):
            i = 2 * kk + d
            gbuf = gbufs[d]
            tbuf = tbufs[d]
            gather_cp(i, d).wait()

            @pl.when(i >= 2)
            def _():
                for cp in out_cps(i - 2, d):
                    cp.wait()

            s, bb = unit_sb(i)
            pvs = [pe_v[s, pl.ds(jc * _L, _L)] for jc in range(_EMB // _L)]

            def transpose_pe(t0, t_v):
                t = 4 * t0
                for dt in range(4):
                    for jc in range(_EMB // _L):
                        val = gbuf[t + dt, pl.ds(jc * _L, _L)] + pvs[jc]
                        plsc.store_scatter(
                            tbuf, [jidx[jc], e8_l, t_v], val)
                    t_v = t_v + ones_v
                return t_v

            lax.fori_loop(0, 32, transpose_pe, zeros_v)

            for cp in out_cps(i, d):
                cp.start()

            @pl.when(i + 2 < _UPW)
            def _():
                gather_cp(i + 2, d).start()
        return carry

    lax.fori_loop(0, _UPW // 2, pair_body, 0)

    for d in range(2):
        for cp in out_cps(_UPW - 2 + d, d):
            cp.wait()


def kernel(x, table):
    xt4 = x.T.reshape(_SEQ // 8, 8, 8, 128).transpose(0, 2, 1, 3)
    xt4 = xt4.reshape(_NUNITS, 128).astype(jnp.int32)
    pe = jnp.asarray(_PE)
    tt = table.T
    packed = _pack(tt, tt)
    t2v = packed.reshape(2 * _P, _EMB)
    out5 = _emb_kernel(xt4, t2v, pe)
    return out5.transpose(2, 4, 0, 1, 3).reshape(_BATCH, _SEQ, _EMB)


# speedup vs baseline: 2.1660x; 1.0616x over previous
import functools

import numpy as np
import jax
import jax.numpy as jnp
from jax import lax
from jax.experimental import pallas as pl
from jax.experimental.pallas import tpu as pltpu
from jax.experimental.pallas import tpu_sc as plsc

_VOCAB = 1000000
_EMB = 64
_SEQ = 200
_BATCH = 1024

_NC = 2
_NS = 16
_L = 16
_NW = _NC * _NS

_P = 512000
_MB = 12800

_NUNITS = (_SEQ // 8) * 8 * 8
_UPW = _NUNITS // _NW


def _pos_encoding():
    pos = np.arange(_SEQ, dtype=np.float32)[:, None]
    i = np.arange(0, _EMB, 2, dtype=np.float32)
    div = np.power(10000.0, i / _EMB)
    pe = np.zeros((_SEQ, _EMB), np.float32)
    pe[:, 0::2] = np.sin(pos / div)
    pe[:, 1::2] = np.cos(pos / div)
    return pe


_PE = _pos_encoding()


def _pack_body(in0_ref, in1_ref, o_ref):
    i0 = lax.broadcasted_iota(jnp.int32, (_EMB, _EMB), 0)
    i1 = lax.broadcasted_iota(jnp.int32, (_EMB, _EMB), 1)
    ident = (i0 == i1).astype(jnp.float32)
    dn = (((0,), (0,)), ((), ()))
    t0 = lax.dot_general(
        in0_ref[...], ident, dn, preferred_element_type=jnp.float32)
    t1 = lax.dot_general(
        in1_ref[...], ident, dn, preferred_element_type=jnp.float32)
    o_ref[...] = jnp.concatenate([t0, t1], axis=1)


_N_IN_BLOCKS = _VOCAB // _MB

_pack = pl.pallas_call(
    _pack_body,
    grid=(_P // _MB,),
    in_specs=[
        pl.BlockSpec((_EMB, _MB), lambda q: (0, q)),
        pl.BlockSpec((_EMB, _MB),
                     lambda q: (0, jnp.minimum(q + _P // _MB, _N_IN_BLOCKS))),
    ],
    out_specs=pl.BlockSpec((_MB, 2 * _EMB), lambda q: (q, 0)),
    out_shape=jax.ShapeDtypeStruct((_P, 2 * _EMB), jnp.float32),
)

_mesh = plsc.VectorSubcoreMesh(core_axis_name="c", subcore_axis_name="s")


@functools.partial(
    pl.kernel,
    mesh=_mesh,
    out_type=jax.ShapeDtypeStruct((_SEQ, 8, 8, 8, 128), jnp.float32),
    compiler_params=pltpu.CompilerParams(use_tc_tiling_on_sc=False,
                                         needs_layout_passes=False),
    scratch_types=[
        pltpu.VMEM((_UPW, 128), jnp.int32),
        pltpu.VMEM((_SEQ, _EMB), jnp.float32),
        pltpu.VMEM((128, _EMB), jnp.float32),
        pltpu.VMEM((128, _EMB), jnp.float32),
        pltpu.VMEM((8, 8, 129), jnp.float32),
        pltpu.VMEM((8, 8, 129), jnp.float32),
        pltpu.SemaphoreType.DMA,
        pltpu.SemaphoreType.DMA,
        pltpu.SemaphoreType.DMA,
        pltpu.SemaphoreType.DMA,
    ],
)
def _emb_kernel(x_hbm, table_hbm, pe_hbm, out_hbm,
                idx_v, pe_v, gbuf0, gbuf1, tbuf0, tbuf1,
                gsem0, gsem1, osem0, osem1):
    wid = lax.axis_index("s") * _NC + lax.axis_index("c")
    ubase = wid * _UPW

    pltpu.sync_copy(x_hbm.at[pl.ds(ubase, _UPW)], idx_v)
    pltpu.sync_copy(pe_hbm, pe_v)

    def remap_row(k, carry):
        for c in range(128 // _L):
            v = idx_v[k, pl.ds(c * _L, _L)]
            idx_v[k, pl.ds(c * _L, _L)] = jnp.where(
                v >= _P, 2 * (v - _P) + 1, 2 * v)
        return carry

    lax.fori_loop(0, _UPW, remap_row, 0)

    gbufs = (gbuf0, gbuf1)
    tbufs = (tbuf0, tbuf1)
    gsems = (gsem0, gsem1)
    osems = (osem0, osem1)

    lane = lax.broadcasted_iota(jnp.int32, (_L,), 0)
    e8_l = lane & 7
    je_half = lane >> 3
    jidx = [je_half + 2 * jc for jc in range(_EMB // _L)]
    ones_v = jnp.full((_L,), 1, jnp.int32)
    zeros_v = jnp.zeros((_L,), jnp.int32)

    def gather_cp(i, p):
        return pltpu.make_async_copy(
            table_hbm.at[idx_v.at[i]], gbufs[p], gsems[p])

    def unit_sb(i):
        v = ubase + i
        s = (v >> 6) * 8 + (v & 7)
        bb = (v >> 3) & 7
        return s, bb

    def out_cps(i, p):
        s, bb = unit_sb(i)
        return [pltpu.make_async_copy(tbufs[p].at[:, :, pl.ds(0, 128)],
                                      out_hbm.at[s, :, bb], osems[p])]

    gather_cp(0, 0).start()
    gather_cp(1, 1).start()

    def pair_body(kk, carry):
        for d in range(2):
            i = 2 * kk + d
            gbuf = gbufs[d]
            tbuf = tbufs[d]
            gather_cp(i, d).wait()

            @pl.when(i >= 2)
            def _():
                for cp in out_cps(i - 2, d):
                    cp.wait()

            s, bb = unit_sb(i)
            pvs = [pe_v[s, pl.ds(jc * _L, _L)] for jc in range(_EMB // _L)]

            def transpose_pe(t0, t_v):
                t = 4 * t0
                for dt in range(4):
                    for jc in range(_EMB // _L):
                        val = gbuf[t + dt, pl.ds(jc * _L, _L)] + pvs[jc]
                        plsc.store_scatter(
                            tbuf, [jidx[jc], e8_l, t_v], val)
                    t_v = t_v + ones_v
                return t_v

            lax.fori_loop(0, 32, transpose_pe, zeros_v)

            for cp in out_cps(i, d):
                cp.start()

            @pl.when(i + 2 < _UPW)
            def _():
                gather_cp(i + 2, d).start()
        return carry

    lax.fori_loop(0, _UPW // 2, pair_body, 0)

    for d in range(2):
        for cp in out_cps(_UPW - 2 + d, d):
            cp.wait()


def kernel(x, table):
    xt4 = x.T.reshape(_SEQ // 8, 8, 8, 128).transpose(0, 2, 1, 3)
    xt4 = xt4.reshape(_NUNITS, 128).astype(jnp.int32)
    pe = jnp.asarray(_PE)
    tt = table.T
    packed = _pack(tt, tt)
    t2v = packed.reshape(2 * _P, _EMB)
    out5 = _emb_kernel(xt4, t2v, pe)
    return out5.transpose(2, 4, 0, 1, 3).reshape(_BATCH, _SEQ, _EMB)
